# baseline (speedup 1.0000x reference)
import functools

import jax
import jax.numpy as jnp
from jax import lax
from jax.experimental import pallas as pl
from jax.experimental.pallas import tpu as pltpu
from jax.experimental.pallas import tpu_sc as plsc

INPUT_DIM = 2048
SIDE_DIM = 2048
CONTEXT_DIM = 4
NUM_CTX = 2 ** CONTEXT_DIM
BATCH = 4096
KB = 256

NC = 1
NS = 16
NW = NC * NS
BPW = BATCH // NW
LANES = 16
ROW = NUM_CTX * BPW + BPW


def _tc_body(side_ref, logit_ref, v_ref, b_ref, w_ref, bc_ref, buf_ref,
             proj_acc, dots_acc):
    k = pl.program_id(0)
    pv = jnp.dot(v_ref[...], side_ref[...],
                 preferred_element_type=jnp.float32)
    pw = jnp.dot(w_ref[...], logit_ref[...],
                 preferred_element_type=jnp.float32)

    @pl.when(k == 0)
    def _():
        proj_acc[...] = pv
        dots_acc[...] = pw

    @pl.when(k > 0)
    def _():
        proj_acc[...] += pv
        dots_acc[...] += pw

    @pl.when(k == pl.num_programs(0) - 1)
    def _():
        bits = (proj_acc[...] > b_ref[...]).astype(jnp.float32)
        ctxf = jnp.sum(bits * bc_ref[...], axis=0)
        dots = dots_acc[...]
        merged = jnp.concatenate(
            [dots.reshape(NUM_CTX, NW, BPW).swapaxes(0, 1).reshape(NW, NUM_CTX * BPW),
             ctxf.reshape(NW, BPW)], axis=1)
        buf_ref[...] = merged.reshape(NW, 1, ROW)


def _sc_route(buf_hbm, out_hbm, buf_v, out_v):
    wid = lax.axis_index("s") * NC + lax.axis_index("c")
    base = wid * BPW
    pltpu.sync_copy(buf_hbm.at[wid, 0], buf_v)
    for i in range(BPW // LANES):
        rows = buf_v[pl.ds(NUM_CTX * BPW + i * LANES, LANES)].astype(jnp.int32)
        acc = jnp.zeros((LANES,), jnp.float32)
        for k in range(NUM_CTX):
            val = buf_v[pl.ds(k * BPW + i * LANES, LANES)]
            acc = jnp.where(rows == k, val, acc)
        out_v[pl.ds(i * LANES, LANES)] = acc
    pltpu.sync_copy(out_v, out_hbm.at[pl.ds(base, BPW)])


def kernel(logit_previous, side_information, v, b, weights, boolean_converter):
    grid = SIDE_DIM // KB
    buf = pl.pallas_call(
        _tc_body,
        grid=(grid,),
        in_specs=[
            pl.BlockSpec((KB, BATCH), lambda k: (k, 0)),
            pl.BlockSpec((KB, BATCH), lambda k: (k, 0)),
            pl.BlockSpec((CONTEXT_DIM, KB), lambda k: (0, k)),
            pl.BlockSpec((CONTEXT_DIM, 1), lambda k: (0, 0)),
            pl.BlockSpec((NUM_CTX, KB), lambda k: (0, k)),
            pl.BlockSpec((CONTEXT_DIM, 1), lambda k: (0, 0)),
        ],
        out_specs=pl.BlockSpec((NW, 1, ROW), lambda k: (0, 0, 0)),
        out_shape=jax.ShapeDtypeStruct((NW, 1, ROW), jnp.float32),
        scratch_shapes=[
            pltpu.VMEM((CONTEXT_DIM, BATCH), jnp.float32),
            pltpu.VMEM((NUM_CTX, BATCH), jnp.float32),
        ],
    )(side_information, logit_previous, v, b, weights, boolean_converter)

    route = functools.partial(
        pl.kernel,
        mesh=plsc.VectorSubcoreMesh(core_axis_name="c", subcore_axis_name="s",
                                    num_cores=NC),
        out_type=jax.ShapeDtypeStruct((BATCH,), jnp.float32),
        scratch_types=[
            pltpu.VMEM((ROW,), jnp.float32),
            pltpu.VMEM((BPW,), jnp.float32),
        ],
    )(_sc_route)
    return route(buf)


# speedup vs baseline: 1.0275x; 1.0275x over previous
import functools

import jax
import jax.numpy as jnp
from jax import lax
from jax.experimental import pallas as pl
from jax.experimental.pallas import tpu as pltpu
from jax.experimental.pallas import tpu_sc as plsc

INPUT_DIM = 2048
SIDE_DIM = 2048
CONTEXT_DIM = 4
NUM_CTX = 2 ** CONTEXT_DIM
BATCH = 4096
BB = 512

NC = 1
NS = 16
NW = NC * NS
BPW = BATCH // NW
LANES = 16
ROW = NUM_CTX * BPW + BPW


def _tc_body(side_ref, logit_ref, v_ref, b_ref, w_ref, bc_ref, buf_ref):
    proj = jnp.dot(v_ref[...], side_ref[...],
                   preferred_element_type=jnp.float32)
    bits = (proj > b_ref[...]).astype(jnp.float32)
    ctxf = jnp.sum(bits * bc_ref[...], axis=0)
    dots = jnp.dot(w_ref[...], logit_ref[...],
                   preferred_element_type=jnp.float32)
    wpb = BB // BPW
    merged = jnp.concatenate(
        [dots.reshape(NUM_CTX, wpb, BPW).swapaxes(0, 1).reshape(wpb, NUM_CTX * BPW),
         ctxf.reshape(wpb, BPW)], axis=1)
    buf_ref[...] = merged.reshape(wpb, 1, ROW)


def _sc_route(buf_hbm, out_hbm, buf_v, out_v):
    wid = lax.axis_index("s") * NC + lax.axis_index("c")
    base = wid * BPW
    pltpu.sync_copy(buf_hbm.at[wid, 0], buf_v)
    for i in range(BPW // LANES):
        rows = buf_v[pl.ds(NUM_CTX * BPW + i * LANES, LANES)].astype(jnp.int32)
        acc = jnp.zeros((LANES,), jnp.float32)
        for k in range(NUM_CTX):
            val = buf_v[pl.ds(k * BPW + i * LANES, LANES)]
            acc = jnp.where(rows == k, val, acc)
        out_v[pl.ds(i * LANES, LANES)] = acc
    pltpu.sync_copy(out_v, out_hbm.at[pl.ds(base, BPW)])


def kernel(logit_previous, side_information, v, b, weights, boolean_converter):
    grid = BATCH // BB
    buf = pl.pallas_call(
        _tc_body,
        grid=(grid,),
        in_specs=[
            pl.BlockSpec((SIDE_DIM, BB), lambda i: (0, i)),
            pl.BlockSpec((INPUT_DIM, BB), lambda i: (0, i)),
            pl.BlockSpec((CONTEXT_DIM, SIDE_DIM), lambda i: (0, 0)),
            pl.BlockSpec((CONTEXT_DIM, 1), lambda i: (0, 0)),
            pl.BlockSpec((NUM_CTX, INPUT_DIM), lambda i: (0, 0)),
            pl.BlockSpec((CONTEXT_DIM, 1), lambda i: (0, 0)),
        ],
        out_specs=pl.BlockSpec((BB // BPW, 1, ROW), lambda i: (i, 0, 0)),
        out_shape=jax.ShapeDtypeStruct((NW, 1, ROW), jnp.float32),
    )(side_information, logit_previous, v, b, weights, boolean_converter)

    route = functools.partial(
        pl.kernel,
        mesh=plsc.VectorSubcoreMesh(core_axis_name="c", subcore_axis_name="s",
                                    num_cores=NC),
        out_type=jax.ShapeDtypeStruct((BATCH,), jnp.float32),
        scratch_types=[
            pltpu.VMEM((ROW,), jnp.float32),
            pltpu.VMEM((BPW,), jnp.float32),
        ],
    )(_sc_route)
    return route(buf)
